# baseline (speedup 1.0000x reference)
import functools

import jax
import jax.numpy as jnp
from jax import lax
from jax.experimental import pallas as pl
from jax.experimental.pallas import tpu as pltpu
from jax.experimental.pallas import tpu_sc as plsc

_ALPHA = 0.25
_B = 16384
_T = 200
_NC = 2
_NS = 16
_NW = _NC * _NS
_RPW = _B // _NW
_G = 16
_NGRP = _RPW // _G
_LN2 = 0.6931471805599453


def _vlog(x):
    bits = lax.bitcast_convert_type(x, jnp.int32)
    e = jnp.right_shift(bits, 23) - 127
    m = lax.bitcast_convert_type(
        jnp.bitwise_or(jnp.bitwise_and(bits, 0x007FFFFF), 0x3F800000), jnp.float32)
    c = m > 1.4142135
    m2 = jnp.where(c, 0.5 * m, m)
    ef = (e + jnp.where(c, 1, 0)).astype(jnp.float32)
    t = (m2 - 1.0) / (m2 + 1.0)
    t2 = t * t
    p = 2.0 * t * (1.0 + t2 * (1.0 / 3.0 + t2 * (0.2 + t2 * (1.0 / 7.0))))
    return _LN2 * ef + p


def _strip_exponent(prod, e_sum):
    bits = lax.bitcast_convert_type(prod, jnp.int32)
    e_sum = e_sum + (jnp.right_shift(bits, 23) - 127)
    m = lax.bitcast_convert_type(
        jnp.bitwise_or(jnp.bitwise_and(bits, 0x007FFFFF), 0x3F800000), jnp.float32)
    return m, e_sum


def _sc_body(yp_hbm, y_hbm, st_hbm, out_hbm, buf, yv, sv, outv):
    wid = lax.axis_index("s") * _NC + lax.axis_index("c")
    row0 = wid * _RPW
    lane_off = lax.iota(jnp.int32, _G) * _T
    ones = jnp.ones((_G,), jnp.float32)

    def group_body(g, acc):
        r0 = row0 + g * _G
        pltpu.sync_copy(yp_hbm.at[pl.ds(r0 * _T, _G * _T)], buf)
        pltpu.sync_copy(y_hbm.at[pl.ds(r0, _G)], yv)
        pltpu.sync_copy(st_hbm.at[pl.ds(r0, _G)], sv)
        y_vec = yv[...]
        wu = jnp.where(sv[...] == 1, 1.0, 0.0)

        prod0 = ones
        prod1 = ones
        e_sum = jnp.zeros((_G,), jnp.int32)
        p_sel = ones
        l_sel = ones
        for t in range(_T):
            v = plsc.load_gather(buf, [lane_off + t])
            om = 1.0 - v
            f = jnp.where(y_vec >= t, om, ones)
            if t % 2 == 0:
                prod0 = prod0 * f
            else:
                prod1 = prod1 * f
            meq = y_vec == t
            p_sel = jnp.where(meq, v, p_sel)
            l_sel = jnp.where(meq, om, l_sel)
            if t % 32 == 31:
                prod0, e_sum = _strip_exponent(prod0, e_sum)
                prod1, e_sum = _strip_exponent(prod1, e_sum)

        prod0, e_sum = _strip_exponent(prod0, e_sum)
        prod1, e_sum = _strip_exponent(prod1, e_sum)
        s_y = _LN2 * e_sum.astype(jnp.float32) + _vlog(prod0 * prod1)

        logs = _vlog(p_sel)
        s_ym1 = jnp.where(y_vec >= 1, s_y - _vlog(l_sel), 0.0)
        lz = wu * (logs + s_ym1)
        lu = wu * _vlog(1.0 - jnp.exp(s_y))
        lc = (1.0 - wu) * s_y
        return acc + _ALPHA * (-lz) + (1.0 - _ALPHA) * (-(lu + lc))

    acc = lax.fori_loop(0, _NGRP, group_body, jnp.zeros((_G,), jnp.float32))
    outv[...] = acc
    pltpu.sync_copy(outv, out_hbm.at[pl.ds(wid * _G, _G)])


@jax.jit
def kernel(y_pred, y, status):
    y32 = y.astype(jnp.int32)
    st32 = status.astype(jnp.int32)
    mesh = plsc.VectorSubcoreMesh(core_axis_name="c", subcore_axis_name="s")
    partials = functools.partial(
        pl.kernel,
        mesh=mesh,
        compiler_params=pltpu.CompilerParams(
            needs_layout_passes=False, use_tc_tiling_on_sc=False),
        out_type=jax.ShapeDtypeStruct((_NW * _G,), jnp.float32),
        scratch_types=[
            pltpu.VMEM((_G * _T,), jnp.float32),
            pltpu.VMEM((_G,), jnp.int32),
            pltpu.VMEM((_G,), jnp.int32),
            pltpu.VMEM((_G,), jnp.float32),
        ],
    )(_sc_body)(y_pred.reshape(-1), y32, st32)
    return jnp.sum(partials)


# speedup vs baseline: 1.1890x; 1.1890x over previous
import functools

import jax
import jax.numpy as jnp
from jax import lax
from jax.experimental import pallas as pl
from jax.experimental.pallas import tpu as pltpu
from jax.experimental.pallas import tpu_sc as plsc

_ALPHA = 0.25
_B = 16384
_T = 200
_NC = 2
_NS = 16
_NW = _NC * _NS
_RPW = _B // _NW
_G = 16
_CR = 32
_GPC = _CR // _G
_NCH = _RPW // _CR
_LN2 = 0.6931471805599453


def _vlog(x):
    bits = lax.bitcast_convert_type(x, jnp.int32)
    e = jnp.right_shift(bits, 23) - 127
    m = lax.bitcast_convert_type(
        jnp.bitwise_or(jnp.bitwise_and(bits, 0x007FFFFF), 0x3F800000), jnp.float32)
    c = m > 1.4142135
    m2 = jnp.where(c, 0.5 * m, m)
    ef = (e + jnp.where(c, 1, 0)).astype(jnp.float32)
    t = (m2 - 1.0) / (m2 + 1.0)
    t2 = t * t
    p = 2.0 * t * (1.0 + t2 * (1.0 / 3.0 + t2 * (0.2 + t2 * (1.0 / 7.0))))
    return _LN2 * ef + p


def _strip_exponent(prod, e_sum):
    bits = lax.bitcast_convert_type(prod, jnp.int32)
    e_sum = e_sum + (jnp.right_shift(bits, 23) - 127)
    m = lax.bitcast_convert_type(
        jnp.bitwise_or(jnp.bitwise_and(bits, 0x007FFFFF), 0x3F800000), jnp.float32)
    return m, e_sum


def _sc_body(yp_hbm, y_hbm, st_hbm, out_hbm, buf0, buf1, yv, sv, outv,
             sem0, sem1):
    wid = lax.axis_index("s") * _NC + lax.axis_index("c")
    row0 = wid * _RPW
    lane_off = lax.iota(jnp.int32, _G) * _T
    ones = jnp.ones((_G,), jnp.float32)

    pltpu.sync_copy(y_hbm.at[pl.ds(row0, _RPW)], yv)
    pltpu.sync_copy(st_hbm.at[pl.ds(row0, _RPW)], sv)

    bufs = (buf0, buf1)
    sems = (sem0, sem1)

    def start(c, b):
        pltpu.async_copy(
            yp_hbm.at[pl.ds((row0 + c * _CR) * _T, _CR * _T)], bufs[b], sems[b])

    def wait(c, b):
        pltpu.make_async_copy(
            yp_hbm.at[pl.ds((row0 + c * _CR) * _T, _CR * _T)], bufs[b], sems[b]
        ).wait()

    def group(buf, c, gi, acc):
        base = c * _CR + gi * _G
        y_vec = yv[pl.ds(base, _G)]
        wu = jnp.where(sv[pl.ds(base, _G)] == 1, 1.0, 0.0)

        prod0 = ones
        prod1 = ones
        e_sum = jnp.zeros((_G,), jnp.int32)
        p_sel = ones
        l_sel = ones
        goff = gi * _G * _T
        for t in range(_T):
            v = plsc.load_gather(buf, [lane_off + (goff + t)])
            om = 1.0 - v
            f = jnp.where(y_vec >= t, om, ones)
            if t % 2 == 0:
                prod0 = prod0 * f
            else:
                prod1 = prod1 * f
            meq = y_vec == t
            p_sel = jnp.where(meq, v, p_sel)
            l_sel = jnp.where(meq, om, l_sel)
            if t % 32 == 31:
                prod0, e_sum = _strip_exponent(prod0, e_sum)
                prod1, e_sum = _strip_exponent(prod1, e_sum)

        prod0, e_sum = _strip_exponent(prod0, e_sum)
        prod1, e_sum = _strip_exponent(prod1, e_sum)
        s_y = _LN2 * e_sum.astype(jnp.float32) + _vlog(prod0 * prod1)

        logs = _vlog(p_sel)
        s_ym1 = jnp.where(y_vec >= 1, s_y - _vlog(l_sel), 0.0)
        lz = wu * (logs + s_ym1)
        lu = wu * _vlog(1.0 - jnp.exp(s_y))
        lc = (1.0 - wu) * s_y
        return acc + _ALPHA * (-lz) + (1.0 - _ALPHA) * (-(lu + lc))

    start(0, 0)

    def chunk_pair(i, acc):
        c = i * 2
        for b in range(2):
            wait(c + b, b)

            @pl.when(c + b + 1 < _NCH)
            def _():
                start(c + b + 1, 1 - b)

            for gi in range(_GPC):
                acc = group(bufs[b], c + b, gi, acc)
        return acc

    acc = lax.fori_loop(0, _NCH // 2, chunk_pair, jnp.zeros((_G,), jnp.float32))
    outv[...] = acc
    pltpu.sync_copy(outv, out_hbm.at[pl.ds(wid * _G, _G)])


@jax.jit
def kernel(y_pred, y, status):
    y32 = y.astype(jnp.int32)
    st32 = status.astype(jnp.int32)
    mesh = plsc.VectorSubcoreMesh(core_axis_name="c", subcore_axis_name="s")
    partials = functools.partial(
        pl.kernel,
        mesh=mesh,
        compiler_params=pltpu.CompilerParams(
            needs_layout_passes=False, use_tc_tiling_on_sc=False),
        out_type=jax.ShapeDtypeStruct((_NW * _G,), jnp.float32),
        scratch_types=[
            pltpu.VMEM((_CR * _T,), jnp.float32),
            pltpu.VMEM((_CR * _T,), jnp.float32),
            pltpu.VMEM((_RPW,), jnp.int32),
            pltpu.VMEM((_RPW,), jnp.int32),
            pltpu.VMEM((_G,), jnp.float32),
            pltpu.SemaphoreType.DMA,
            pltpu.SemaphoreType.DMA,
        ],
    )(_sc_body)(y_pred.reshape(-1), y32, st32)
    return jnp.sum(partials)


# speedup vs baseline: 3.1244x; 2.6277x over previous
import jax
import jax.numpy as jnp
from jax.experimental import pallas as pl

_ALPHA = 0.25
_B = 16384
_T = 200
_BK = 4096


def _body(yp_ref, ys_ref, out_ref):
    i = pl.program_id(0)
    p = yp_ref[...]
    packed = ys_ref[...]
    yb = jnp.bitwise_and(packed, 0x3FFFFFFF)
    wu = jnp.right_shift(packed, 30).astype(jnp.float32)

    t = jax.lax.broadcasted_iota(jnp.int32, p.shape, 1)
    l1m = jnp.log(1.0 - p)
    m_le = (t <= yb).astype(jnp.float32)
    m_eq = (t == yb).astype(jnp.float32)

    s_y = jnp.sum(l1m * m_le, axis=1, keepdims=True)
    l1m_y = jnp.sum(l1m * m_eq, axis=1, keepdims=True)
    p_y = jnp.sum(p * m_eq, axis=1, keepdims=True)

    s_ym1 = jnp.where(yb >= 1, s_y - l1m_y, 0.0)
    lz = wu * (jnp.log(p_y) + s_ym1)
    lu = wu * jnp.log(1.0 - jnp.exp(s_y))
    lc = (1.0 - wu) * s_y
    total = _ALPHA * (-jnp.sum(lz)) + (1.0 - _ALPHA) * (-(jnp.sum(lu) + jnp.sum(lc)))

    @pl.when(i == 0)
    def _init():
        out_ref[...] = jnp.zeros_like(out_ref)

    out_ref[...] += total


@jax.jit
def kernel(y_pred, y, status):
    packed = (y.astype(jnp.int32)
              | (status.astype(jnp.int32) << 30))[:, None]
    grid = _B // _BK
    out = pl.pallas_call(
        _body,
        grid=(grid,),
        in_specs=[
            pl.BlockSpec((_BK, _T), lambda i: (i, 0)),
            pl.BlockSpec((_BK, 1), lambda i: (i, 0)),
        ],
        out_specs=pl.BlockSpec((1, 1), lambda i: (0, 0)),
        out_shape=jax.ShapeDtypeStruct((1, 1), jnp.float32),
    )(y_pred, packed)
    return out[0, 0]


# speedup vs baseline: 3.6361x; 1.1638x over previous
import jax
import jax.numpy as jnp
from jax.experimental import pallas as pl

_ALPHA = 0.25
_B = 16384
_T = 200
_BK = 4096


def _body(yp_ref, ys_ref, out_ref):
    i = pl.program_id(0)
    p = yp_ref[...]
    packed = ys_ref[pl.ds(i * _BK, _BK)].reshape(_BK, 1)
    yb = jnp.bitwise_and(packed, 0x3FFFFFFF)
    wu = jnp.right_shift(packed, 30).astype(jnp.float32)

    t = jax.lax.broadcasted_iota(jnp.int32, p.shape, 1)
    l1m = jnp.log(1.0 - p)
    m_le = (t <= yb).astype(jnp.float32)
    m_eq = (t == yb).astype(jnp.float32)

    s_y = jnp.sum(l1m * m_le, axis=1, keepdims=True)
    l1m_y = jnp.sum(l1m * m_eq, axis=1, keepdims=True)
    p_y = jnp.sum(p * m_eq, axis=1, keepdims=True)

    s_ym1 = jnp.where(yb >= 1, s_y - l1m_y, 0.0)
    lz = wu * (jnp.log(p_y) + s_ym1)
    lu = wu * jnp.log(1.0 - jnp.exp(s_y))
    lc = (1.0 - wu) * s_y
    total = _ALPHA * (-jnp.sum(lz)) + (1.0 - _ALPHA) * (-(jnp.sum(lu) + jnp.sum(lc)))

    @pl.when(i == 0)
    def _init():
        out_ref[...] = jnp.zeros_like(out_ref)

    out_ref[...] += total


@jax.jit
def kernel(y_pred, y, status):
    packed = y.astype(jnp.int32) | (status.astype(jnp.int32) << 30)
    grid = _B // _BK
    out = pl.pallas_call(
        _body,
        grid=(grid,),
        in_specs=[
            pl.BlockSpec((_BK, _T), lambda i: (i, 0)),
            pl.BlockSpec((_B,), lambda i: (0,)),
        ],
        out_specs=pl.BlockSpec((1, 1), lambda i: (0, 0)),
        out_shape=jax.ShapeDtypeStruct((1, 1), jnp.float32),
    )(y_pred, packed)
    return out[0, 0]
